# baseline (speedup 1.0000x reference)
import functools

import jax
import jax.numpy as jnp
from jax import lax
from jax.experimental import pallas as pl
from jax.experimental.pallas import tpu as pltpu
from jax.experimental.pallas import tpu_sc as plsc

B, S, D = 4, 4096, 1024
MAX_ROWS = 64
N = B * S
NC, NS = 2, 16
NW = NC * NS
TPW = N // NW
C = 8
NCHUNK = TPW // C
NPAIR = NCHUNK // 2
LANES = 16
GROUPS = D // LANES


def _body(tokens_hbm, idx_hbm, table_hbm, out_hbm,
          idx_v, table_v, tin0, tin1, tout0, tout1,
          sem_t, in_sem0, in_sem1, out_sem0, out_sem1):
    wid = lax.axis_index("s") * NC + lax.axis_index("c")
    base = wid * TPW

    tbl_copy = pltpu.async_copy(table_hbm, table_v, sem_t)
    pltpu.sync_copy(idx_hbm.at[pl.ds(base, TPW)], idx_v)
    pltpu.async_copy(tokens_hbm.at[pl.ds(base + 0 * C, C)], tin0, in_sem0)
    pltpu.async_copy(tokens_hbm.at[pl.ds(base + 1 * C, C)], tin1, in_sem1)
    tbl_copy.wait()

    iota = lax.iota(jnp.int32, LANES)

    def wait_in(tin, sem):
        pltpu.make_async_copy(tokens_hbm.at[pl.ds(base, C)], tin, sem).wait()

    def wait_out(tout, sem):
        pltpu.make_async_copy(tout, out_hbm.at[pl.ds(base, C)], sem).wait()

    def compute(idx_vec, half, tin, tout):
        def row_body(i, carry):
            lane = jnp.full((LANES, 1), half * C + i, dtype=jnp.int32)
            dnums = lax.GatherDimensionNumbers(
                offset_dims=(), collapsed_slice_dims=(0,), start_index_map=(0,))
            row = lax.gather(idx_vec, lane, dnums, (1,),
                             mode=lax.GatherScatterMode.PROMISE_IN_BOUNDS)
            row_base = row * D
            for j in range(GROUPS):
                sl = pl.ds(j * LANES, LANES)
                g = plsc.load_gather(table_v, [row_base + (iota + j * LANES)])
                tout[i, sl] = tin[i, sl] + g
            return carry

        lax.fori_loop(0, C, row_body, 0)

    def pair(k, first, last):
        idx_vec = idx_v[pl.ds(k * 2 * C, LANES)]
        c0 = 2 * k
        c1 = 2 * k + 1
        wait_in(tin0, in_sem0)
        if not first:
            wait_out(tout0, out_sem0)
        compute(idx_vec, 0, tin0, tout0)
        pltpu.async_copy(tout0, out_hbm.at[pl.ds(base + c0 * C, C)], out_sem0)
        if not last:
            pltpu.async_copy(tokens_hbm.at[pl.ds(base + (c0 + 2) * C, C)],
                             tin0, in_sem0)
        wait_in(tin1, in_sem1)
        if not first:
            wait_out(tout1, out_sem1)
        compute(idx_vec, 1, tin1, tout1)
        pltpu.async_copy(tout1, out_hbm.at[pl.ds(base + c1 * C, C)], out_sem1)
        if not last:
            pltpu.async_copy(tokens_hbm.at[pl.ds(base + (c1 + 2) * C, C)],
                             tin1, in_sem1)

    pair(0, True, False)

    def mid(k, carry):
        pair(k, False, False)
        return carry

    lax.fori_loop(1, NPAIR - 1, mid, 0)
    pair(NPAIR - 1, False, True)

    wait_out(tout0, out_sem0)
    wait_out(tout1, out_sem1)


@jax.jit
def _run(tokens2d, idx1d, table_flat):
    mesh = plsc.VectorSubcoreMesh(core_axis_name="c", subcore_axis_name="s")
    f = pl.kernel(
        _body,
        out_type=jax.ShapeDtypeStruct((N, D), jnp.float32),
        mesh=mesh,
        compiler_params=pltpu.CompilerParams(needs_layout_passes=False),
        scratch_types=[
            pltpu.VMEM((TPW,), jnp.int32),
            pltpu.VMEM((MAX_ROWS * D,), jnp.float32),
            pltpu.VMEM((C, D), jnp.float32),
            pltpu.VMEM((C, D), jnp.float32),
            pltpu.VMEM((C, D), jnp.float32),
            pltpu.VMEM((C, D), jnp.float32),
            pltpu.SemaphoreType.DMA,
            pltpu.SemaphoreType.DMA,
            pltpu.SemaphoreType.DMA,
            pltpu.SemaphoreType.DMA,
            pltpu.SemaphoreType.DMA,
        ],
    )
    return f(tokens2d, idx1d, table_flat)


def kernel(tokens, example_positions, pos_table):
    tokens2d = tokens.reshape(N, D)
    idx1d = example_positions.reshape(N).astype(jnp.int32)
    out = _run(tokens2d, idx1d, pos_table.reshape(MAX_ROWS * D))
    return out.reshape(B, S, D)


# speedup vs baseline: 1.4841x; 1.4841x over previous
import jax
import jax.numpy as jnp
from jax import lax
from jax.experimental import pallas as pl
from jax.experimental.pallas import tpu as pltpu
from jax.experimental.pallas import tpu_sc as plsc

B, S, D = 4, 4096, 1024
MAX_ROWS = 64
N = B * S
NC, NS = 2, 16
NW = NC * NS
TPW = N // NW
C = 16
NCHUNK = TPW // C
NPAIR = NCHUNK // 2
LANES = 16
GROUPS = D // LANES


def _body(tokens_hbm, idx_hbm, table_hbm, out_hbm,
          idx_v, tin0, tin1, emb0, emb1, tout0, tout1,
          in_sem0, in_sem1, g_sem0, g_sem1, out_sem0, out_sem1):
    wid = lax.axis_index("s") * NC + lax.axis_index("c")
    base = wid * TPW

    pltpu.sync_copy(idx_hbm.at[pl.ds(base, TPW)], idx_v)

    def start_in(c, tin, sem):
        pltpu.async_copy(tokens_hbm.at[pl.ds(base + c * C, C)], tin, sem)

    def start_gather(c, emb, sem):
        pltpu.async_copy(table_hbm.at[idx_v.at[pl.ds(c * C, C)]], emb, sem)

    def start_out(c, tout, sem):
        pltpu.async_copy(tout, out_hbm.at[pl.ds(base + c * C, C)], sem)

    def wait_in(tin, sem):
        pltpu.make_async_copy(tokens_hbm.at[pl.ds(base, C)], tin, sem).wait()

    def wait_gather(emb, sem):
        pltpu.make_async_copy(table_hbm.at[pl.ds(0, C)], emb, sem).wait()

    def wait_out(tout, sem):
        pltpu.make_async_copy(tout, out_hbm.at[pl.ds(base, C)], sem).wait()

    def compute(tin, emb, tout):
        def row_body(i, carry):
            for j in range(GROUPS):
                sl = pl.ds(j * LANES, LANES)
                tout[i, sl] = tin[i, sl] + emb[i, sl]
            return carry

        lax.fori_loop(0, C, row_body, 0)

    start_in(0, tin0, in_sem0)
    start_gather(0, emb0, g_sem0)
    start_in(1, tin1, in_sem1)
    start_gather(1, emb1, g_sem1)

    def half(c, tin, emb, tout, in_sem, g_sem, out_sem, first, last):
        wait_in(tin, in_sem)
        wait_gather(emb, g_sem)
        if not first:
            wait_out(tout, out_sem)
        compute(tin, emb, tout)
        start_out(c, tout, out_sem)
        if not last:
            start_in(c + 2, tin, in_sem)
            start_gather(c + 2, emb, g_sem)

    def pair(k, first, last):
        half(2 * k, tin0, emb0, tout0, in_sem0, g_sem0, out_sem0, first, last)
        half(2 * k + 1, tin1, emb1, tout1, in_sem1, g_sem1, out_sem1,
             first, last)

    def mid(k, carry):
        pair(k, False, False)
        return carry

    pair(0, True, False)
    lax.fori_loop(1, NPAIR - 1, mid, 0)
    pair(NPAIR - 1, False, True)

    wait_out(tout0, out_sem0)
    wait_out(tout1, out_sem1)


@jax.jit
def _run(tokens2d, idx1d, table):
    mesh = plsc.VectorSubcoreMesh(core_axis_name="c", subcore_axis_name="s")
    f = pl.kernel(
        _body,
        out_type=jax.ShapeDtypeStruct((N, D), jnp.float32),
        mesh=mesh,
        scratch_types=[
            pltpu.VMEM((TPW,), jnp.int32),
            pltpu.VMEM((C, D), jnp.float32),
            pltpu.VMEM((C, D), jnp.float32),
            pltpu.VMEM((C, D), jnp.float32),
            pltpu.VMEM((C, D), jnp.float32),
            pltpu.VMEM((C, D), jnp.float32),
            pltpu.VMEM((C, D), jnp.float32),
            pltpu.SemaphoreType.DMA,
            pltpu.SemaphoreType.DMA,
            pltpu.SemaphoreType.DMA,
            pltpu.SemaphoreType.DMA,
            pltpu.SemaphoreType.DMA,
            pltpu.SemaphoreType.DMA,
        ],
    )
    return f(tokens2d, idx1d, table)


def kernel(tokens, example_positions, pos_table):
    tokens2d = tokens.reshape(N, D)
    idx1d = example_positions.reshape(N).astype(jnp.int32)
    out = _run(tokens2d, idx1d, pos_table)
    return out.reshape(B, S, D)


# speedup vs baseline: 1.4920x; 1.0053x over previous
import jax
import jax.numpy as jnp
from jax import lax
from jax.experimental import pallas as pl
from jax.experimental.pallas import tpu as pltpu
from jax.experimental.pallas import tpu_sc as plsc

B, S, D = 4, 4096, 1024
MAX_ROWS = 64
N = B * S
NC, NS = 2, 16
NW = NC * NS
TPW = N // NW
C = 16
NBUF = 4
NCHUNK = TPW // C
NGROUP = NCHUNK // NBUF


def _body(tokens_hbm, idx_hbm, table_hbm, out_hbm,
          idx_v, tin0, tin1, tin2, tin3,
          in_sem0, in_sem1, in_sem2, in_sem3,
          g_sem0, g_sem1, g_sem2, g_sem3,
          out_sem0, out_sem1, out_sem2, out_sem3):
    wid = lax.axis_index("s") * NC + lax.axis_index("c")
    base = wid * TPW

    tins = (tin0, tin1, tin2, tin3)
    in_sems = (in_sem0, in_sem1, in_sem2, in_sem3)
    g_sems = (g_sem0, g_sem1, g_sem2, g_sem3)
    out_sems = (out_sem0, out_sem1, out_sem2, out_sem3)

    pltpu.sync_copy(idx_hbm.at[pl.ds(base, TPW)], idx_v)

    def start_in(c, b):
        pltpu.async_copy(tokens_hbm.at[pl.ds(base + c * C, C)], tins[b],
                         in_sems[b])

    def start_gadd(c, b):
        pltpu.async_copy(table_hbm.at[idx_v.at[pl.ds(c * C, C)]], tins[b],
                         g_sems[b], add=True)

    def start_out(c, b):
        pltpu.async_copy(tins[b], out_hbm.at[pl.ds(base + c * C, C)],
                         out_sems[b])

    def wait_in(b):
        pltpu.make_async_copy(tokens_hbm.at[pl.ds(base, C)], tins[b],
                              in_sems[b]).wait()

    def wait_gadd(b):
        pltpu.make_async_copy(table_hbm.at[pl.ds(0, C)], tins[b],
                              g_sems[b]).wait()

    def wait_out(b):
        pltpu.make_async_copy(tins[b], out_hbm.at[pl.ds(base, C)],
                              out_sems[b]).wait()

    for b in range(NBUF):
        start_in(b, b)

    def group(g, first, last):
        for b in range(NBUF):
            wait_in(b)
            start_gadd(g * NBUF + b, b)
        for b in range(NBUF):
            wait_gadd(b)
            start_out(g * NBUF + b, b)
        if not last:
            for b in range(NBUF):
                wait_out(b)
                start_in((g + 1) * NBUF + b, b)

    def mid(g, carry):
        group(g, False, False)
        return carry

    group(0, True, False)
    lax.fori_loop(1, NGROUP - 1, mid, 0)
    group(NGROUP - 1, False, True)

    for b in range(NBUF):
        wait_out(b)


@jax.jit
def _run(tokens2d, idx1d, table):
    mesh = plsc.VectorSubcoreMesh(core_axis_name="c", subcore_axis_name="s")
    f = pl.kernel(
        _body,
        out_type=jax.ShapeDtypeStruct((N, D), jnp.float32),
        mesh=mesh,
        compiler_params=pltpu.CompilerParams(needs_layout_passes=False),
        scratch_types=[
            pltpu.VMEM((TPW,), jnp.int32),
            pltpu.VMEM((C, D), jnp.float32),
            pltpu.VMEM((C, D), jnp.float32),
            pltpu.VMEM((C, D), jnp.float32),
            pltpu.VMEM((C, D), jnp.float32),
            pltpu.SemaphoreType.DMA,
            pltpu.SemaphoreType.DMA,
            pltpu.SemaphoreType.DMA,
            pltpu.SemaphoreType.DMA,
            pltpu.SemaphoreType.DMA,
            pltpu.SemaphoreType.DMA,
            pltpu.SemaphoreType.DMA,
            pltpu.SemaphoreType.DMA,
            pltpu.SemaphoreType.DMA,
            pltpu.SemaphoreType.DMA,
            pltpu.SemaphoreType.DMA,
            pltpu.SemaphoreType.DMA,
        ],
    )
    return f(tokens2d, idx1d, table)


def kernel(tokens, example_positions, pos_table):
    tokens2d = tokens.reshape(N, D)
    idx1d = example_positions.reshape(N).astype(jnp.int32)
    out = _run(tokens2d, idx1d, pos_table)
    return out.reshape(B, S, D)
